# Initial kernel scaffold; baseline (speedup 1.0000x reference)
import jax
import jax.numpy as jnp
from jax.experimental import pallas as pl


def kernel(x, l, params):
    raise NotImplementedError("write your pallas kernel here")


# speedup vs baseline: 1.2489x; 1.2489x over previous
import functools

import jax
import jax.numpy as jnp
from jax import lax
from jax.experimental import pallas as pl
from jax.experimental.pallas import tpu as pltpu

_NEG = -3.0e38


def _lrelu(h):
    return jnp.where(h >= 0, h, 0.2 * h)


def _knn_body(xr_ref, xa_ref, o_ref, d_scr, *, T, N, Kk, Kpad):
    xr = xr_ref[0]
    xa = xa_ref[0]
    inner = jnp.dot(xr, xa, preferred_element_type=jnp.float32)
    sqr = jnp.sum(xr * xr, axis=1, keepdims=True)
    sqa = jnp.sum(xa * xa, axis=0, keepdims=True)
    d_scr[...] = 2.0 * inner - sqr - sqa

    lane = lax.broadcasted_iota(jnp.int32, (T, N), 1)
    kiota = lax.broadcasted_iota(jnp.int32, (T, Kpad), 1)

    def body(i, acc):
        d = d_scr[...]
        m = jnp.max(d, axis=1, keepdims=True)
        cand = jnp.where(d >= m, lane, N)
        amin = jnp.min(cand, axis=1, keepdims=True)
        d_scr[...] = jnp.where(lane == amin, _NEG, d)
        return jnp.where(kiota == i, amin, acc)

    acc = lax.fori_loop(0, Kk, body, jnp.zeros((T, Kpad), jnp.int32))
    acc = jnp.where(kiota < Kk, acc, acc[:, 0:1])
    o_ref[0] = acc


def _knn_topk(xfl, xff, k, kpad, tile):
    B_, N_, Cp = xfl.shape
    T = min(tile, N_)
    grid = (B_, N_ // T)
    return pl.pallas_call(
        functools.partial(_knn_body, T=T, N=N_, Kk=k, Kpad=kpad),
        grid=grid,
        in_specs=[
            pl.BlockSpec((1, T, Cp), lambda b, i: (b, i, 0)),
            pl.BlockSpec((1, Cp, N_), lambda b, i: (b, 0, 0)),
        ],
        out_specs=pl.BlockSpec((1, T, kpad), lambda b, i: (b, i, 0)),
        out_shape=jax.ShapeDtypeStruct((B_, N_, kpad), jnp.int32),
        scratch_shapes=[pltpu.VMEM((T, N_), jnp.float32)],
    )(xfl, xff)


def _edge_body(xr_ref, nbr_ref, w1_ref, g1_ref, b1_ref, w2_ref, g2_ref,
               b2_ref, o_ref, *, T, Kn, C):
    ctr = xr_ref[0][:, :C]
    nbr = nbr_ref[0][:, :C]
    ctrb = jnp.broadcast_to(ctr[:, None, :], (T, Kn, C)).reshape(T * Kn, C)
    feat = jnp.concatenate([nbr - ctrb, ctrb], axis=-1)
    h = jnp.dot(feat, w1_ref[...], preferred_element_type=jnp.float32)
    h = _lrelu(h * g1_ref[...] + b1_ref[...])
    h = jnp.dot(h, w2_ref[...], preferred_element_type=jnp.float32)
    h = _lrelu(h * g2_ref[...] + b2_ref[...])
    o_ref[0] = jnp.max(h.reshape(T, Kn, h.shape[-1]), axis=1)


def _edge_conv(xfl, nbr, layers, kpad, tile):
    B_, N_, Cp = xfl.shape
    p1, p2 = layers
    C = p1['W'].shape[1] // 2
    Cout = p2['W'].shape[0]
    T = min(tile, N_)
    grid = (B_, N_ // T)
    args = (
        xfl, nbr,
        p1['W'].T, p1['g'][None, :], p1['b'][None, :],
        p2['W'].T, p2['g'][None, :], p2['b'][None, :],
    )
    full = lambda s: pl.BlockSpec(s, lambda b, i: tuple(0 for _ in s))
    return pl.pallas_call(
        functools.partial(_edge_body, T=T, Kn=kpad, C=C),
        grid=grid,
        in_specs=[
            pl.BlockSpec((1, T, Cp), lambda b, i: (b, i, 0)),
            pl.BlockSpec((1, T * kpad, Cp), lambda b, i: (b, i, 0)),
            full(p1['W'].T.shape), full((1, p1['W'].shape[0])),
            full((1, p1['W'].shape[0])),
            full(p2['W'].T.shape), full((1, Cout)), full((1, Cout)),
        ],
        out_specs=pl.BlockSpec((1, T, Cout), lambda b, i: (b, i, 0)),
        out_shape=jax.ShapeDtypeStruct((B_, N_, Cout), jnp.float32),
    )(*args)


def _mlp_body(x_ref, w_ref, g_ref, b_ref, o_ref, *, act):
    h = jnp.dot(x_ref[0], w_ref[...], preferred_element_type=jnp.float32)
    h = h * g_ref[...] + b_ref[...]
    if act:
        h = _lrelu(h)
    o_ref[0] = h


def _mlp(x, p, act=True, tile=512):
    B_, N_, Cin = x.shape
    Cout = p['W'].shape[0]
    T = min(tile, N_)
    grid = (B_, N_ // T)
    full = lambda s: pl.BlockSpec(s, lambda b, i: tuple(0 for _ in s))
    return pl.pallas_call(
        functools.partial(_mlp_body, act=act),
        grid=grid,
        in_specs=[
            pl.BlockSpec((1, T, Cin), lambda b, i: (b, i, 0)),
            full((Cin, Cout)), full((1, Cout)), full((1, Cout)),
        ],
        out_specs=pl.BlockSpec((1, T, Cout), lambda b, i: (b, i, 0)),
        out_shape=jax.ShapeDtypeStruct((B_, N_, Cout), jnp.float32),
    )(x, p['W'].T, p['g'][None, :], p['b'][None, :])


def _mlpmax_body(x_ref, w_ref, g_ref, b_ref, o_ref):
    j = pl.program_id(1)
    h = jnp.dot(x_ref[0], w_ref[...], preferred_element_type=jnp.float32)
    h = _lrelu(h * g_ref[...] + b_ref[...])
    m = jnp.max(h, axis=0, keepdims=True)

    @pl.when(j == 0)
    def _():
        o_ref[0] = m

    @pl.when(j > 0)
    def _():
        o_ref[0] = jnp.maximum(o_ref[0], m)


def _mlp_max(x, p, tile=512):
    B_, N_, Cin = x.shape
    Cout = p['W'].shape[0]
    T = min(tile, N_)
    grid = (B_, N_ // T)
    full = lambda s: pl.BlockSpec(s, lambda b, i: tuple(0 for _ in s))
    return pl.pallas_call(
        _mlpmax_body,
        grid=grid,
        in_specs=[
            pl.BlockSpec((1, T, Cin), lambda b, i: (b, i, 0)),
            full((Cin, Cout)), full((1, Cout)), full((1, Cout)),
        ],
        out_specs=pl.BlockSpec((1, 1, Cout), lambda b, i: (b, 0, 0)),
        out_shape=jax.ShapeDtypeStruct((B_, 1, Cout), jnp.float32),
    )(x, p['W'].T, p['g'][None, :], p['b'][None, :])[:, 0]


def _unpool_body(dst_ref, src_ref, xc_ref, o_ref, *, T, M):
    dst = dst_ref[0]
    src = src_ref[0]
    d = jnp.zeros((T, M), jnp.float32)
    for c in range(3):
        diff = dst[:, c:c + 1] - src[c:c + 1, :]
        d = d + diff * diff
    lane = lax.broadcasted_iota(jnp.int32, (T, M), 1)
    dmin = jnp.min(d, axis=1, keepdims=True)
    cand = jnp.where(d <= dmin, lane, M)
    amin = jnp.min(cand, axis=1, keepdims=True)
    onehot = (lane == amin).astype(jnp.float32)
    o_ref[0] = jnp.dot(onehot, xc_ref[0], preferred_element_type=jnp.float32)


def _unpool(dstfl, srcff, xc, tile=512):
    B_, N_, Cp = dstfl.shape
    M = srcff.shape[2]
    C = xc.shape[2]
    T = min(tile, N_)
    grid = (B_, N_ // T)
    return pl.pallas_call(
        functools.partial(_unpool_body, T=T, M=M),
        grid=grid,
        in_specs=[
            pl.BlockSpec((1, T, Cp), lambda b, i: (b, i, 0)),
            pl.BlockSpec((1, Cp, M), lambda b, i: (b, 0, 0)),
            pl.BlockSpec((1, M, C), lambda b, i: (b, 0, 0)),
        ],
        out_specs=pl.BlockSpec((1, T, C), lambda b, i: (b, i, 0)),
        out_shape=jax.ShapeDtypeStruct((B_, N_, C), jnp.float32),
    )(dstfl, srcff, xc)


def _gather_rows(table, idx):
    return jnp.take_along_axis(table, idx[:, :, None], axis=1)


_K = 40
_NPOINT = 2048


def kernel(x, l, params):
    B_ = x.shape[0]
    N0 = x.shape[2]

    xff0 = jnp.concatenate(
        [x, jnp.zeros((B_, 5, N0), jnp.float32)], axis=1)
    xfl0 = jnp.transpose(xff0, (0, 2, 1))

    n1, n2, n3 = N0 // 4, N0 // 16, N0 // 64
    xff1, xfl1 = xff0[:, :, :n1], xfl0[:, :n1]
    xff2, xfl2 = xff0[:, :, :n2], xfl0[:, :n2]
    xff3, xfl3 = xff0[:, :, :n3], xfl0[:, :n3]

    idx0 = _knn_topk(xfl0, xff0, _K, _K, tile=512)
    nbr0 = _gather_rows(xfl0, idx0.reshape(B_, -1))
    x0 = _edge_conv(xfl0, nbr0, params['ec0'], _K, tile=256)
    xt0 = _mlp_max(x0, params['pn0'][0])

    x0p = x0[:, :n1]
    idx1 = _knn_topk(x0p, jnp.transpose(x0p, (0, 2, 1)), _K, _K, tile=512)
    nbr1 = _gather_rows(x0p, idx1.reshape(B_, -1))
    x1 = _edge_conv(x0p, nbr1, params['ec1'], _K, tile=128)
    xt1 = _mlp_max(x1, params['pn1'][0])

    x1p = x1[:, :n2]
    idx2 = _knn_topk(x1p, jnp.transpose(x1p, (0, 2, 1)), _K, _K, tile=128)
    nbr2 = _gather_rows(x1p, idx2.reshape(B_, -1))
    x2 = _edge_conv(x1p, nbr2, params['ec2'], _K, tile=128)
    xt2 = _mlp_max(x2, params['pn2'][0])

    x2p = x2[:, :n3]
    idx3 = _knn_topk(x2p, jnp.transpose(x2p, (0, 2, 1)), _K // 2, 24, tile=32)
    nbr3 = _gather_rows(x2p, idx3.reshape(B_, -1))
    x3 = _edge_conv(x2p, nbr3, params['ec3'], 24, tile=32)
    xt3 = _mlp_max(x3, params['pn3'][0])

    xcg = jnp.maximum(jnp.maximum(xt0, xt1), jnp.maximum(xt2, xt3))
    lv = _mlp(l[:, None, :], params['label_conv'][0])[:, 0]
    glob = jnp.concatenate([xcg, lv], axis=-1)

    xc = jnp.concatenate(
        [jnp.broadcast_to(glob[:, None, :], (B_, n3, glob.shape[-1])), x3],
        axis=-1)
    xc = _mlp(xc, params['pn4'][0], tile=32)

    xc = _unpool(xfl2, xff3, xc)
    xc = jnp.concatenate([xc, x2], axis=-1)
    xc = _mlp(xc, params['pn5'][0], tile=128)

    xc = _unpool(xfl1, xff2, xc)
    xc = jnp.concatenate([xc, x1], axis=-1)
    xc = _mlp(xc, params['pn6'][0], tile=512)

    xc = _unpool(xfl0, xff1, xc)
    xc = jnp.concatenate([xc, x0], axis=-1)
    xc = _mlp(xc, params['pn7'][0], tile=512)

    w8 = params['conv8']
    out = _mlp(xc, {'W': w8, 'g': jnp.ones((w8.shape[0],), jnp.float32),
                    'b': jnp.zeros((w8.shape[0],), jnp.float32)},
               act=False, tile=512)
    out = jnp.transpose(out, (0, 2, 1))

    node1 = x[:, :, :n1]
    node2 = x[:, :, :n2]
    node3 = x[:, :, :n3]
    ret1 = jnp.arange(n1)
    ret2 = jnp.arange(n2)
    ret3 = jnp.arange(n3)
    return (out, ret1, ret2, ret3, node1, node2, node3, node1, node2, node3)
